# baseline (speedup 1.0000x reference)
import jax
import jax.numpy as jnp
from jax import lax
from jax.experimental import pallas as pl
from jax.experimental.pallas import tpu as pltpu
from jax.experimental.pallas import tpu_sc as plsc

B, S, D, F, E, TOP_K = 1, 2048, 768, 3072, 8, 2
T = B * S
A = T * TOP_K
BT = 128
NB = A // BT + E
NSLOT = NB * BT
NBMAX = 64
NC, NS = 2, 16
NW = NC * NS
TPW = T // NW
CH = 8
GW = 128
CS = T // CH


def _router_body(x_ref, wr_ref, s0_ref, s1_ref, g0_ref, g1_ref, bexp_ref):
    xf = x_ref[...]
    logits = jnp.dot(xf, wr_ref[...], preferred_element_type=jnp.float32)
    probs = jax.nn.softmax(logits, axis=-1)
    i8 = lax.broadcasted_iota(jnp.int32, probs.shape, 1)
    m0 = jnp.max(probs, axis=-1, keepdims=True)
    e0 = jnp.min(jnp.where(probs == m0, i8, E), axis=-1, keepdims=True)
    probsm = jnp.where(i8 == e0, -jnp.inf, probs)
    m1 = jnp.max(probsm, axis=-1, keepdims=True)
    e1 = jnp.min(jnp.where(probsm == m1, i8, E), axis=-1, keepdims=True)
    ssum = m0 + m1
    oh0 = (i8 == e0).astype(jnp.float32)
    oh1 = (i8 == e1).astype(jnp.float32)
    m = oh0 + oh1

    r = lax.broadcasted_iota(jnp.int32, (CS, CS), 0)
    c = lax.broadcasted_iota(jnp.int32, (CS, CS), 1)
    ltri = (c < r).astype(jnp.float32)
    pcs, tots = [], []
    for ci in range(CH):
        blk = lax.slice_in_dim(m, ci * CS, (ci + 1) * CS, axis=0)
        pcs.append(jnp.dot(ltri, blk, preferred_element_type=jnp.float32))
        tots.append(jnp.sum(blk, axis=0, keepdims=True))
    tot = jnp.concatenate(tots, axis=0)
    r8 = lax.broadcasted_iota(jnp.int32, (CH, CH), 0)
    c8 = lax.broadcasted_iota(jnp.int32, (CH, CH), 1)
    ltri8 = (c8 < r8).astype(jnp.float32)
    choff = jnp.dot(ltri8, tot, preferred_element_type=jnp.float32)
    p = jnp.concatenate(
        [pcs[ci] + choff[ci:ci + 1] for ci in range(CH)], axis=0)

    counts = choff[CH - 1:CH] + tot[CH - 1:CH]
    padded = jnp.floor((counts + (BT - 1)) * (1.0 / BT)) * BT
    re = lax.broadcasted_iota(jnp.int32, (E, E), 0)
    ce = lax.broadcasted_iota(jnp.int32, (E, E), 1)
    utri = (re < ce).astype(jnp.float32)
    offs = jnp.dot(padded, utri, preferred_element_type=jnp.float32)

    rank0 = jnp.sum(oh0 * p, axis=-1, keepdims=True)
    rank1 = jnp.sum(oh1 * p, axis=-1, keepdims=True)
    off0 = jnp.sum(oh0 * offs, axis=-1, keepdims=True)
    off1 = jnp.sum(oh1 * offs, axis=-1, keepdims=True)
    s0_ref[...] = (rank0 + off0).astype(jnp.int32)
    s1_ref[...] = (rank1 + off1).astype(jnp.int32)
    zrow = jnp.zeros((1, GW), jnp.float32)
    g0_ref[...] = m0 / ssum + zrow
    g1_ref[...] = m1 / ssum + zrow

    ends = (offs + padded) * (1.0 / BT)
    bv = lax.broadcasted_iota(jnp.int32, (NBMAX, E), 0).astype(jnp.float32)
    acc = jnp.sum((bv >= ends).astype(jnp.int32), axis=-1, keepdims=True)
    bexp_ref[...] = jnp.minimum(acc, E - 1)


def _router(xf, Wr):
    return pl.pallas_call(
        _router_body,
        out_shape=(jax.ShapeDtypeStruct((T, 1), jnp.int32),
                   jax.ShapeDtypeStruct((T, 1), jnp.int32),
                   jax.ShapeDtypeStruct((T, GW), jnp.float32),
                   jax.ShapeDtypeStruct((T, GW), jnp.float32),
                   jax.ShapeDtypeStruct((NBMAX, 1), jnp.int32)),
    )(xf, Wr)


def _dispatch_body(x_hbm, s0_hbm, s1_hbm, g0_hbm, g1_hbm, xg_hbm, gs_hbm,
                   s0_v, s1_v, xbuf, g0buf, g1buf,
                   semx, sem0, sem1, sem2, sem3):
    wid = lax.axis_index("s") * NC + lax.axis_index("c")
    off = pl.multiple_of(wid * TPW, TPW)
    pltpu.sync_copy(s0_hbm.at[pl.ds(off, TPW)], s0_v)
    pltpu.sync_copy(s1_hbm.at[pl.ds(off, TPW)], s1_v)
    pltpu.async_copy(x_hbm.at[pl.ds(off, TPW)], xbuf, semx).wait()
    pltpu.sync_copy(g0_hbm.at[pl.ds(off, TPW)], g0buf)
    pltpu.sync_copy(g1_hbm.at[pl.ds(off, TPW)], g1buf)
    c0 = pltpu.async_copy(xbuf, xg_hbm.at[s0_v], sem0)
    c1 = pltpu.async_copy(xbuf, xg_hbm.at[s1_v], sem1)
    c2 = pltpu.async_copy(g0buf, gs_hbm.at[s0_v], sem2)
    c3 = pltpu.async_copy(g1buf, gs_hbm.at[s1_v], sem3)
    c0.wait()
    c1.wait()
    c2.wait()
    c3.wait()


def _dispatch(xf, s0, s1, g0, g1):
    mesh = plsc.VectorSubcoreMesh(core_axis_name="c", subcore_axis_name="s")
    return pl.kernel(
        _dispatch_body,
        out_type=(jax.ShapeDtypeStruct((NSLOT, D), jnp.float32),
                  jax.ShapeDtypeStruct((NSLOT, GW), jnp.float32)),
        mesh=mesh,
        scratch_types=[
            pltpu.VMEM((TPW,), jnp.int32),
            pltpu.VMEM((TPW,), jnp.int32),
            pltpu.VMEM((TPW, D), jnp.float32),
            pltpu.VMEM((TPW, GW), jnp.float32),
            pltpu.VMEM((TPW, GW), jnp.float32),
            pltpu.SemaphoreType.DMA,
            pltpu.SemaphoreType.DMA,
            pltpu.SemaphoreType.DMA,
            pltpu.SemaphoreType.DMA,
            pltpu.SemaphoreType.DMA,
        ],
    )(xf, s0, s1, g0, g1)


def _ffn_body(m_ref, xg_ref, w1_ref, b1_ref, w2_ref, b2_ref, gs_ref, yg_ref):
    h = jnp.dot(xg_ref[...], w1_ref[0], preferred_element_type=jnp.float32)
    h = jax.nn.gelu(h + b1_ref[0])
    y = jnp.dot(h, w2_ref[0], preferred_element_type=jnp.float32) + b2_ref[0]
    yg_ref[...] = y * gs_ref[:, 0:1]


def _ffn(bexp, xg, W1, b1r, W2, b2r, gslot):
    return pl.pallas_call(
        _ffn_body,
        grid_spec=pltpu.PrefetchScalarGridSpec(
            num_scalar_prefetch=1,
            grid=(NB,),
            in_specs=[
                pl.BlockSpec((BT, D), lambda b, m: (b, 0)),
                pl.BlockSpec((1, D, F), lambda b, m: (m[b], 0, 0)),
                pl.BlockSpec((1, 1, F), lambda b, m: (m[b], 0, 0)),
                pl.BlockSpec((1, F, D), lambda b, m: (m[b], 0, 0)),
                pl.BlockSpec((1, 1, D), lambda b, m: (m[b], 0, 0)),
                pl.BlockSpec((BT, GW), lambda b, m: (b, 0)),
            ],
            out_specs=pl.BlockSpec((BT, D), lambda b, m: (b, 0)),
        ),
        out_shape=jax.ShapeDtypeStruct((NSLOT, D), jnp.float32),
        compiler_params=pltpu.CompilerParams(
            dimension_semantics=("arbitrary",),
        ),
    )(bexp, xg, W1, b1r, W2, b2r, gslot)


def _combine_body(yg_hbm, s0_hbm, s1_hbm, out_hbm, s0_v, s1_v, buf0, buf1,
                  sem0, sem1):
    wid = lax.axis_index("s") * NC + lax.axis_index("c")
    off = pl.multiple_of(wid * TPW, TPW)
    pltpu.sync_copy(s0_hbm.at[pl.ds(off, TPW)], s0_v)
    pltpu.sync_copy(s1_hbm.at[pl.ds(off, TPW)], s1_v)
    g0 = pltpu.async_copy(yg_hbm.at[s0_v], buf0, sem0)
    g1 = pltpu.async_copy(yg_hbm.at[s1_v], buf1, sem1)
    g0.wait()
    g1.wait()

    def tok(t, c):
        for cc in range(D // 16):
            buf0[t, pl.ds(cc * 16, 16)] = (buf0[t, pl.ds(cc * 16, 16)]
                                           + buf1[t, pl.ds(cc * 16, 16)])
        return c

    lax.fori_loop(0, TPW, tok, 0)
    pltpu.sync_copy(buf0, out_hbm.at[pl.ds(off, TPW)])


def _combine(yg, s0, s1):
    mesh = plsc.VectorSubcoreMesh(core_axis_name="c", subcore_axis_name="s")
    return pl.kernel(
        _combine_body,
        out_type=jax.ShapeDtypeStruct((T, D), jnp.float32),
        mesh=mesh,
        scratch_types=[
            pltpu.VMEM((TPW,), jnp.int32),
            pltpu.VMEM((TPW,), jnp.int32),
            pltpu.VMEM((TPW, D), jnp.float32),
            pltpu.VMEM((TPW, D), jnp.float32),
            pltpu.SemaphoreType.DMA,
            pltpu.SemaphoreType.DMA,
        ],
    )(yg, s0, s1)


@jax.jit
def kernel(x, Wr, W1, b1, W2, b2):
    xf = x.reshape(T, D)
    s0, s1, g0, g1, bexp = _router(xf, Wr)
    s0 = s0.reshape(T)
    s1 = s1.reshape(T)
    xg, gslot = _dispatch(xf, s0, s1, g0, g1)
    yg = _ffn(bexp.reshape(NBMAX), xg, W1, b1.reshape(E, 1, F), W2,
              b2.reshape(E, 1, D), gslot)
    out = _combine(yg, s0, s1)
    return out.reshape(B, S, D)


# speedup vs baseline: 1.1430x; 1.1430x over previous
import jax
import jax.numpy as jnp
from jax import lax
from jax.experimental import pallas as pl
from jax.experimental.pallas import tpu as pltpu
from jax.experimental.pallas import tpu_sc as plsc

B, S, D, F, E, TOP_K = 1, 2048, 768, 3072, 8, 2
T = B * S
A = T * TOP_K
BT = 256
NB = A // BT + E
NSLOT = NB * BT
NBMAX = 32
NC, NS = 2, 16
NW = NC * NS
TPW = T // NW
CH = 8
GW = 128
CS = T // CH


def _router_body(x_ref, wr_ref, s0_ref, s1_ref, g0_ref, g1_ref, bexp_ref,
                 bact_ref):
    xf = x_ref[...]
    logits = jnp.dot(xf, wr_ref[...], preferred_element_type=jnp.float32)
    i8 = lax.broadcasted_iota(jnp.int32, logits.shape, 1)
    t0 = jnp.max(logits, axis=-1, keepdims=True)
    e0 = jnp.min(jnp.where(logits == t0, i8, E), axis=-1, keepdims=True)
    lm = jnp.where(i8 == e0, -jnp.inf, logits)
    t1 = jnp.max(lm, axis=-1, keepdims=True)
    e1 = jnp.min(jnp.where(lm == t1, i8, E), axis=-1, keepdims=True)
    ex = jnp.exp(t1 - t0)
    den = 1.0 + ex
    oh0 = (i8 == e0).astype(jnp.float32)
    oh1 = (i8 == e1).astype(jnp.float32)
    m = oh0 + oh1

    r = lax.broadcasted_iota(jnp.int32, (CS, CS), 0)
    c = lax.broadcasted_iota(jnp.int32, (CS, CS), 1)
    ltri = (c < r).astype(jnp.float32)
    pcs, tots = [], []
    for ci in range(CH):
        blk = lax.slice_in_dim(m, ci * CS, (ci + 1) * CS, axis=0)
        pcs.append(jnp.dot(ltri, blk, preferred_element_type=jnp.float32))
        tots.append(jnp.sum(blk, axis=0, keepdims=True))
    tot = jnp.concatenate(tots, axis=0)
    r8 = lax.broadcasted_iota(jnp.int32, (CH, CH), 0)
    c8 = lax.broadcasted_iota(jnp.int32, (CH, CH), 1)
    ltri8 = (c8 < r8).astype(jnp.float32)
    choff = jnp.dot(ltri8, tot, preferred_element_type=jnp.float32)
    p = jnp.concatenate(
        [pcs[ci] + choff[ci:ci + 1] for ci in range(CH)], axis=0)

    counts = choff[CH - 1:CH] + tot[CH - 1:CH]
    padded = jnp.floor((counts + (BT - 1)) * (1.0 / BT)) * BT
    re = lax.broadcasted_iota(jnp.int32, (E, E), 0)
    ce = lax.broadcasted_iota(jnp.int32, (E, E), 1)
    utri = (re < ce).astype(jnp.float32)
    offs = jnp.dot(padded, utri, preferred_element_type=jnp.float32)

    rank0 = jnp.sum(oh0 * p, axis=-1, keepdims=True)
    rank1 = jnp.sum(oh1 * p, axis=-1, keepdims=True)
    off0 = jnp.sum(oh0 * offs, axis=-1, keepdims=True)
    off1 = jnp.sum(oh1 * offs, axis=-1, keepdims=True)
    s0_ref[...] = (rank0 + off0).astype(jnp.int32)
    s1_ref[...] = (rank1 + off1).astype(jnp.int32)
    zrow = jnp.zeros((1, GW), jnp.float32)
    g0_ref[...] = 1.0 / den + zrow
    g1_ref[...] = ex / den + zrow

    ends = (offs + padded) * (1.0 / BT)
    bv = lax.broadcasted_iota(jnp.int32, (NBMAX, E), 0).astype(jnp.float32)
    acc = jnp.sum((bv >= ends).astype(jnp.int32), axis=-1, keepdims=True)
    bexp_ref[...] = jnp.minimum(acc, E - 1)
    nact = ends[0, E - 1]
    bact_ref[...] = (bv[:, 0:1] < nact).astype(jnp.int32)


def _router(xf, Wr):
    return pl.pallas_call(
        _router_body,
        out_shape=(jax.ShapeDtypeStruct((T, 1), jnp.int32),
                   jax.ShapeDtypeStruct((T, 1), jnp.int32),
                   jax.ShapeDtypeStruct((T, GW), jnp.float32),
                   jax.ShapeDtypeStruct((T, GW), jnp.float32),
                   jax.ShapeDtypeStruct((NBMAX, 1), jnp.int32),
                   jax.ShapeDtypeStruct((NBMAX, 1), jnp.int32)),
    )(xf, Wr)


def _dispatch_body(x_hbm, s0_hbm, s1_hbm, g0_hbm, g1_hbm, xg_hbm, gs_hbm,
                   s0_v, s1_v, xbuf, g0buf, g1buf,
                   semx, sem0, sem1, sem2, sem3):
    wid = lax.axis_index("s") * NC + lax.axis_index("c")
    off = pl.multiple_of(wid * TPW, TPW)
    pltpu.sync_copy(s0_hbm.at[pl.ds(off, TPW)], s0_v)
    pltpu.sync_copy(s1_hbm.at[pl.ds(off, TPW)], s1_v)
    pltpu.async_copy(x_hbm.at[pl.ds(off, TPW)], xbuf, semx).wait()
    pltpu.sync_copy(g0_hbm.at[pl.ds(off, TPW)], g0buf)
    pltpu.sync_copy(g1_hbm.at[pl.ds(off, TPW)], g1buf)
    c0 = pltpu.async_copy(xbuf, xg_hbm.at[s0_v], sem0)
    c1 = pltpu.async_copy(xbuf, xg_hbm.at[s1_v], sem1)
    c2 = pltpu.async_copy(g0buf, gs_hbm.at[s0_v], sem2)
    c3 = pltpu.async_copy(g1buf, gs_hbm.at[s1_v], sem3)
    c0.wait()
    c1.wait()
    c2.wait()
    c3.wait()


def _dispatch(xf, s0, s1, g0, g1):
    mesh = plsc.VectorSubcoreMesh(core_axis_name="c", subcore_axis_name="s")
    return pl.kernel(
        _dispatch_body,
        out_type=(jax.ShapeDtypeStruct((NSLOT, D), jnp.float32),
                  jax.ShapeDtypeStruct((NSLOT, GW), jnp.float32)),
        mesh=mesh,
        scratch_types=[
            pltpu.VMEM((TPW,), jnp.int32),
            pltpu.VMEM((TPW,), jnp.int32),
            pltpu.VMEM((TPW, D), jnp.float32),
            pltpu.VMEM((TPW, GW), jnp.float32),
            pltpu.VMEM((TPW, GW), jnp.float32),
            pltpu.SemaphoreType.DMA,
            pltpu.SemaphoreType.DMA,
            pltpu.SemaphoreType.DMA,
            pltpu.SemaphoreType.DMA,
            pltpu.SemaphoreType.DMA,
        ],
    )(xf, s0, s1, g0, g1)


def _ffn_body(m_ref, a_ref, xg_ref, w1_ref, b1_ref, w2_ref, b2_ref, gs_ref,
              yg_ref):
    b = pl.program_id(0)

    @pl.when(a_ref[b] == 1)
    def _():
        h = jnp.dot(xg_ref[...], w1_ref[0],
                    preferred_element_type=jnp.float32)
        h = jax.nn.gelu(h + b1_ref[0])
        y = (jnp.dot(h, w2_ref[0], preferred_element_type=jnp.float32)
             + b2_ref[0])
        yg_ref[...] = y * gs_ref[:, 0:1]


def _ffn(bexp, bact, xg, W1, b1r, W2, b2r, gslot):
    return pl.pallas_call(
        _ffn_body,
        grid_spec=pltpu.PrefetchScalarGridSpec(
            num_scalar_prefetch=2,
            grid=(NB,),
            in_specs=[
                pl.BlockSpec((BT, D), lambda b, m, a: (b, 0)),
                pl.BlockSpec((1, D, F), lambda b, m, a: (m[b], 0, 0)),
                pl.BlockSpec((1, 1, F), lambda b, m, a: (m[b], 0, 0)),
                pl.BlockSpec((1, F, D), lambda b, m, a: (m[b], 0, 0)),
                pl.BlockSpec((1, 1, D), lambda b, m, a: (m[b], 0, 0)),
                pl.BlockSpec((BT, GW), lambda b, m, a: (b, 0)),
            ],
            out_specs=pl.BlockSpec((BT, D), lambda b, m, a: (b, 0)),
        ),
        out_shape=jax.ShapeDtypeStruct((NSLOT, D), jnp.float32),
        compiler_params=pltpu.CompilerParams(
            dimension_semantics=("arbitrary",),
        ),
    )(bexp, bact, xg, W1, b1r, W2, b2r, gslot)


def _combine_body(yg_hbm, s0_hbm, s1_hbm, out_hbm, s0_v, s1_v, buf0, buf1,
                  sem0, sem1):
    wid = lax.axis_index("s") * NC + lax.axis_index("c")
    off = pl.multiple_of(wid * TPW, TPW)
    pltpu.sync_copy(s0_hbm.at[pl.ds(off, TPW)], s0_v)
    pltpu.sync_copy(s1_hbm.at[pl.ds(off, TPW)], s1_v)
    g0 = pltpu.async_copy(yg_hbm.at[s0_v], buf0, sem0)
    g1 = pltpu.async_copy(yg_hbm.at[s1_v], buf1, sem1)
    g0.wait()
    g1.wait()

    def tok(t, c):
        for cc in range(D // 16):
            buf0[t, pl.ds(cc * 16, 16)] = (buf0[t, pl.ds(cc * 16, 16)]
                                           + buf1[t, pl.ds(cc * 16, 16)])
        return c

    lax.fori_loop(0, TPW, tok, 0)
    pltpu.sync_copy(buf0, out_hbm.at[pl.ds(off, TPW)])


def _combine(yg, s0, s1):
    mesh = plsc.VectorSubcoreMesh(core_axis_name="c", subcore_axis_name="s")
    return pl.kernel(
        _combine_body,
        out_type=jax.ShapeDtypeStruct((T, D), jnp.float32),
        mesh=mesh,
        scratch_types=[
            pltpu.VMEM((TPW,), jnp.int32),
            pltpu.VMEM((TPW,), jnp.int32),
            pltpu.VMEM((TPW, D), jnp.float32),
            pltpu.VMEM((TPW, D), jnp.float32),
            pltpu.SemaphoreType.DMA,
            pltpu.SemaphoreType.DMA,
        ],
    )(yg, s0, s1)


@jax.jit
def kernel(x, Wr, W1, b1, W2, b2):
    xf = x.reshape(T, D)
    s0, s1, g0, g1, bexp, bact = _router(xf, Wr)
    s0 = s0.reshape(T)
    s1 = s1.reshape(T)
    xg, gslot = _dispatch(xf, s0, s1, g0, g1)
    yg = _ffn(bexp.reshape(NBMAX), bact.reshape(NBMAX), xg, W1,
              b1.reshape(E, 1, F), W2, b2.reshape(E, 1, D), gslot)
    out = _combine(yg, s0, s1)
    return out.reshape(B, S, D)
